# baseline (speedup 1.0000x reference)
import functools

import jax
import jax.numpy as jnp
from jax import lax
from jax.experimental import pallas as pl
from jax.experimental.pallas import tpu as pltpu
from jax.experimental.pallas import tpu_sc as plsc

NC = 2
NS = 16
L = 16
CH = 128
BLK = 16


def _zero_rows(ref, nrows, width):
  zero = jnp.zeros((L,), jnp.float32)

  def body(i, carry):
    for j in range(width // L):
      ref[i, pl.ds(j * L, L)] = zero
    return carry

  lax.fori_loop(0, nrows, body, 0, unroll=4)


def _sc_mesh():
  return plsc.VectorSubcoreMesh(core_axis_name="c", subcore_axis_name="s")


def _make_deg_kernel(npad, cpt):
  @functools.partial(
      pl.kernel,
      out_type=jax.ShapeDtypeStruct((NC * NS, npad), jnp.float32),
      mesh=_sc_mesh(),
      compiler_params=pltpu.CompilerParams(needs_layout_passes=False),
      scratch_types=[
          pltpu.VMEM((cpt, CH), jnp.int32),
          pltpu.VMEM((npad,), jnp.float32),
      ],
  )
  def deg_kernel(col3, degp, colbuf, hist):
    c = lax.axis_index("c")
    s = lax.axis_index("s")
    wid = c * NS + s

    pltpu.sync_copy(col3.at[wid], colbuf)

    zero = jnp.zeros((L,), jnp.float32)

    def zbody(i, carry):
      hist[pl.ds(i * L, L)] = zero
      return carry

    lax.fori_loop(0, npad // L, zbody, 0, unroll=8)

    one = jnp.full((L,), 1.0, jnp.float32)

    def chunk(j, carry):
      for k in range(CH // L):
        idx = colbuf[j, pl.ds(k * L, L)]
        plsc.addupdate_scatter(hist, [idx], one)
      return carry

    lax.fori_loop(0, cpt, chunk, 0)
    pltpu.sync_copy(hist, degp.at[wid])

  return deg_kernel


def _make_agg_kernel(npad, d, cpt):
  rows_per_tile = npad // NS
  nblk = cpt // BLK

  @functools.partial(
      pl.kernel,
      out_type=jax.ShapeDtypeStruct((NC * npad, d), jnp.float32),
      mesh=_sc_mesh(),
      scratch_types=[
          pltpu.VMEM((2 * BLK, CH), jnp.int32),
          pltpu.VMEM((2 * BLK, CH), jnp.int32),
          pltpu.VMEM((CH, d), jnp.float32),
          pltpu.VMEM((CH, d), jnp.float32),
          pltpu.VMEM_SHARED((npad, d), jnp.float32),
          pltpu.SemaphoreType.DMA,
          pltpu.SemaphoreType.DMA,
          pltpu.SemaphoreType.DMA,
          pltpu.SemaphoreType.DMA,
          pltpu.SemaphoreType.DMA,
      ],
  )
  def agg_kernel(xs_hbm, row4, col4, out, rowring, colring, ra, rb, acc,
                 sga, sgb, ssa, ssb, sia):
    c = lax.axis_index("c")
    s = lax.axis_index("s")
    wid = c * NS + s

    _zero_rows(ra, CH, d)
    zbase = s * rows_per_tile
    for r in range(rows_per_tile // CH):
      pltpu.sync_copy(ra, acc.at[pl.ds(zbase + r * CH, CH)])
    plsc.subcore_barrier()

    pltpu.sync_copy(row4.at[wid * nblk], rowring.at[pl.ds(0, BLK)])
    pltpu.sync_copy(col4.at[wid * nblk], colring.at[pl.ds(0, BLK)])
    pltpu.async_copy(xs_hbm.at[rowring.at[0]], ra, sga)
    pltpu.async_copy(xs_hbm.at[rowring.at[1]], rb, sgb)

    def wait_gather(buf, sem):
      pltpu.make_async_copy(xs_hbm.at[rowring.at[0]], buf, sem).wait()

    def wait_scatter(buf, sem):
      pltpu.make_async_copy(buf, acc.at[colring.at[0]], sem).wait()

    def wait_ring():
      pltpu.make_async_copy(
          row4.at[0], rowring.at[pl.ds(0, BLK)], sia).wait()

    def block(b, carry):
      slot = lax.rem(b, 2)
      base = slot * BLK
      nbase = (1 - slot) * BLK
      for p in range(BLK // 2):
        j = base + 2 * p
        wait_gather(ra, sga)
        pltpu.async_copy(ra, acc.at[colring.at[j]], ssa, add=True)
        wait_gather(rb, sgb)
        pltpu.async_copy(rb, acc.at[colring.at[j + 1]], ssb, add=True)

        if p == 0:
          wait_scatter(ra, ssa)
          wait_scatter(rb, ssb)

          @pl.when(b < nblk - 1)
          def _refill():
            k = wid * nblk + b + 1
            pltpu.async_copy(row4.at[k], rowring.at[pl.ds(nbase, BLK)], sia)
            pltpu.async_copy(col4.at[k], colring.at[pl.ds(nbase, BLK)], sia)

          pltpu.async_copy(xs_hbm.at[rowring.at[j + 2]], ra, sga)
          pltpu.async_copy(xs_hbm.at[rowring.at[j + 3]], rb, sgb)
        elif p < BLK // 2 - 1:
          wait_scatter(ra, ssa)
          pltpu.async_copy(xs_hbm.at[rowring.at[j + 2]], ra, sga)
          wait_scatter(rb, ssb)
          pltpu.async_copy(xs_hbm.at[rowring.at[j + 3]], rb, sgb)
        else:
          wait_scatter(ra, ssa)
          wait_scatter(rb, ssb)

          @pl.when(b < nblk - 1)
          def _next():
            wait_ring()
            wait_ring()
            pltpu.async_copy(xs_hbm.at[rowring.at[nbase]], ra, sga)
            pltpu.async_copy(xs_hbm.at[rowring.at[nbase + 1]], rb, sgb)

      return carry

    lax.fori_loop(0, nblk, block, 0)
    plsc.subcore_barrier()

    pltpu.sync_copy(
        acc.at[pl.ds(s * rows_per_tile, rows_per_tile)],
        out.at[pl.ds(c * npad + s * rows_per_tile, rows_per_tile)])

  return agg_kernel


def _scale_kernel(dp_ref, x_ref, xs_ref, ds_ref):
  deg = jnp.sum(dp_ref[...], axis=0)[:, None]
  dis = jnp.where(deg > 0.0, lax.rsqrt(jnp.maximum(deg, 1e-30)), 0.0)
  xs_ref[...] = dis * x_ref[...]
  ds_ref[...] = jnp.broadcast_to(dis, ds_ref.shape)


def _head_kernel(p0_ref, p1_ref, ds_ref, w_ref, b_ref, o_ref):
  agg = ds_ref[:, :1] * (p0_ref[...] + p1_ref[...])
  out = jnp.dot(agg, w_ref[...], preferred_element_type=jnp.float32)
  o_ref[...] = jnp.maximum(out + b_ref[...], 0.0)


def kernel(x, edge_index, W, b):
  n, d = x.shape
  e = edge_index.shape[1]

  npad = -(-(n + 1) // (NS * L)) * (NS * L)
  cpt = -(-e // (NC * NS * CH))
  cpt = -(-cpt // BLK) * BLK
  epad = NC * NS * cpt * CH

  row = edge_index[0].astype(jnp.int32)
  col = edge_index[1].astype(jnp.int32)
  pad = jnp.full((epad - e,), n, jnp.int32)
  row3 = jnp.concatenate([row, pad]).reshape(NC * NS, cpt, CH)
  col3 = jnp.concatenate([col, pad]).reshape(NC * NS, cpt, CH)
  row4 = row3.reshape(NC * NS * (cpt // BLK), BLK, CH)
  col4 = col3.reshape(NC * NS * (cpt // BLK), BLK, CH)
  x_pad = jnp.zeros((npad, d), x.dtype).at[:n].set(x)

  degp = _make_deg_kernel(npad, cpt)(col3)

  bn = 256
  grid = (npad // bn,)
  xs, ds16 = pl.pallas_call(
      _scale_kernel,
      grid=grid,
      in_specs=[
          pl.BlockSpec((NC * NS, bn), lambda i: (0, i)),
          pl.BlockSpec((bn, d), lambda i: (i, 0)),
      ],
      out_specs=[
          pl.BlockSpec((bn, d), lambda i: (i, 0)),
          pl.BlockSpec((bn, L), lambda i: (i, 0)),
      ],
      out_shape=[
          jax.ShapeDtypeStruct((npad, d), jnp.float32),
          jax.ShapeDtypeStruct((npad, L), jnp.float32),
      ],
  )(degp, x_pad)

  parts = _make_agg_kernel(npad, d, cpt)(xs, row4, col4)

  h_pad = pl.pallas_call(
      _head_kernel,
      grid=grid,
      in_specs=[
          pl.BlockSpec((bn, d), lambda i: (i, 0)),
          pl.BlockSpec((bn, d), lambda i: (i, 0)),
          pl.BlockSpec((bn, L), lambda i: (i, 0)),
          pl.BlockSpec((d, d), lambda i: (0, 0)),
          pl.BlockSpec((1, d), lambda i: (0, 0)),
      ],
      out_specs=pl.BlockSpec((bn, d), lambda i: (i, 0)),
      out_shape=jax.ShapeDtypeStruct((npad, d), jnp.float32),
  )(parts[:npad], parts[npad:], ds16, W, b.reshape(1, d))

  return h_pad[:n]


# speedup vs baseline: 1.0140x; 1.0140x over previous
import functools

import jax
import jax.numpy as jnp
from jax import lax
from jax.experimental import pallas as pl
from jax.experimental.pallas import tpu as pltpu
from jax.experimental.pallas import tpu_sc as plsc

NC = 2
NS = 16
L = 16
CH = 128
BLK = 16


def _zero_rows(ref, nrows, width):
  zero = jnp.zeros((L,), jnp.float32)

  def body(i, carry):
    for j in range(width // L):
      ref[i, pl.ds(j * L, L)] = zero
    return carry

  lax.fori_loop(0, nrows, body, 0, unroll=4)


def _sc_mesh():
  return plsc.VectorSubcoreMesh(core_axis_name="c", subcore_axis_name="s")


def _make_deg_kernel(npad, cpt):
  @functools.partial(
      pl.kernel,
      out_type=jax.ShapeDtypeStruct((NC * NS, npad), jnp.float32),
      mesh=_sc_mesh(),
      compiler_params=pltpu.CompilerParams(needs_layout_passes=False),
      scratch_types=[
          pltpu.VMEM((cpt, CH), jnp.int32),
          pltpu.VMEM((npad,), jnp.float32),
      ],
  )
  def deg_kernel(col3, degp, colbuf, hist):
    c = lax.axis_index("c")
    s = lax.axis_index("s")
    wid = c * NS + s

    pltpu.sync_copy(col3.at[wid], colbuf)

    zero = jnp.zeros((L,), jnp.float32)

    def zbody(i, carry):
      hist[pl.ds(i * L, L)] = zero
      return carry

    lax.fori_loop(0, npad // L, zbody, 0, unroll=8)

    one = jnp.full((L,), 1.0, jnp.float32)

    def chunk(j, carry):
      for k in range(CH // L):
        idx = colbuf[j, pl.ds(k * L, L)]
        plsc.addupdate_scatter(hist, [idx], one)
      return carry

    lax.fori_loop(0, cpt, chunk, 0)
    pltpu.sync_copy(hist, degp.at[wid])

  return deg_kernel


def _make_agg_kernel(npad, d, cpt):
  rows_per_tile = npad // NS
  nblk = cpt // BLK

  @functools.partial(
      pl.kernel,
      out_type=jax.ShapeDtypeStruct((NC * npad, d), jnp.float32),
      mesh=_sc_mesh(),
      scratch_types=[
          pltpu.VMEM((2 * BLK, CH), jnp.int32),
          pltpu.VMEM((2 * BLK, CH), jnp.int32),
          pltpu.VMEM((CH, d), jnp.float32),
          pltpu.VMEM((CH, d), jnp.float32),
          pltpu.VMEM_SHARED((npad, d), jnp.float32),
          pltpu.SemaphoreType.DMA,
          pltpu.SemaphoreType.DMA,
          pltpu.SemaphoreType.DMA,
          pltpu.SemaphoreType.DMA,
          pltpu.SemaphoreType.DMA,
      ],
  )
  def agg_kernel(xs_hbm, row4, col4, out, rowring, colring, ra, rb, acc,
                 sga, sgb, ssa, ssb, sia):
    c = lax.axis_index("c")
    s = lax.axis_index("s")
    wid = c * NS + s

    _zero_rows(ra, CH, d)
    zbase = s * rows_per_tile
    for r in range(rows_per_tile // CH):
      pltpu.sync_copy(ra, acc.at[pl.ds(zbase + r * CH, CH)])
    plsc.subcore_barrier()

    pltpu.sync_copy(row4.at[wid * nblk], rowring.at[pl.ds(0, BLK)])
    pltpu.sync_copy(col4.at[wid * nblk], colring.at[pl.ds(0, BLK)])
    pltpu.async_copy(xs_hbm.at[rowring.at[0]], ra, sga)

    def wait_gather(buf, sem):
      pltpu.make_async_copy(xs_hbm.at[rowring.at[0]], buf, sem).wait()

    def wait_ring():
      pltpu.make_async_copy(
          row4.at[0], rowring.at[pl.ds(0, BLK)], sia).wait()

    def block(b, carry):
      slot = lax.rem(b, 2)
      base = slot * BLK
      nbase = (1 - slot) * BLK

      @pl.when(b < nblk - 1)
      def _refill():
        k = wid * nblk + b + 1
        pltpu.async_copy(row4.at[k], rowring.at[pl.ds(nbase, BLK)], sia)
        pltpu.async_copy(col4.at[k], colring.at[pl.ds(nbase, BLK)], sia)

      bufs = ((ra, sga), (rb, sgb))
      for p in range(BLK):
        j = base + p
        cur, csem = bufs[p % 2]
        nxt, nsem = bufs[(p + 1) % 2]
        wait_gather(cur, csem)
        if p < BLK - 1:
          pltpu.async_copy(xs_hbm.at[rowring.at[j + 1]], nxt, nsem)
        else:
          @pl.when(b < nblk - 1)
          def _next():
            wait_ring()
            wait_ring()
            pltpu.async_copy(xs_hbm.at[rowring.at[nbase]], nxt, nsem)
        pltpu.sync_copy(cur, acc.at[colring.at[j]], add=True)
      return carry

    lax.fori_loop(0, nblk, block, 0)
    plsc.subcore_barrier()

    pltpu.sync_copy(
        acc.at[pl.ds(s * rows_per_tile, rows_per_tile)],
        out.at[pl.ds(c * npad + s * rows_per_tile, rows_per_tile)])

  return agg_kernel


def _scale_kernel(dp_ref, x_ref, xs_ref, ds_ref):
  deg = jnp.sum(dp_ref[...], axis=0)[:, None]
  dis = jnp.where(deg > 0.0, lax.rsqrt(jnp.maximum(deg, 1e-30)), 0.0)
  xs_ref[...] = dis * x_ref[...]
  ds_ref[...] = jnp.broadcast_to(dis, ds_ref.shape)


def _head_kernel(p0_ref, p1_ref, ds_ref, w_ref, b_ref, o_ref):
  agg = ds_ref[:, :1] * (p0_ref[...] + p1_ref[...])
  out = jnp.dot(agg, w_ref[...], preferred_element_type=jnp.float32)
  o_ref[...] = jnp.maximum(out + b_ref[...], 0.0)


def kernel(x, edge_index, W, b):
  n, d = x.shape
  e = edge_index.shape[1]

  npad = -(-(n + 1) // (NS * L)) * (NS * L)
  cpt = -(-e // (NC * NS * CH))
  cpt = -(-cpt // BLK) * BLK
  epad = NC * NS * cpt * CH

  row = edge_index[0].astype(jnp.int32)
  col = edge_index[1].astype(jnp.int32)
  pad = jnp.full((epad - e,), n, jnp.int32)
  row3 = jnp.concatenate([row, pad]).reshape(NC * NS, cpt, CH)
  col3 = jnp.concatenate([col, pad]).reshape(NC * NS, cpt, CH)
  row4 = row3.reshape(NC * NS * (cpt // BLK), BLK, CH)
  col4 = col3.reshape(NC * NS * (cpt // BLK), BLK, CH)
  x_pad = jnp.zeros((npad, d), x.dtype).at[:n].set(x)

  degp = _make_deg_kernel(npad, cpt)(col3)

  bn = 256
  grid = (npad // bn,)
  xs, ds16 = pl.pallas_call(
      _scale_kernel,
      grid=grid,
      in_specs=[
          pl.BlockSpec((NC * NS, bn), lambda i: (0, i)),
          pl.BlockSpec((bn, d), lambda i: (i, 0)),
      ],
      out_specs=[
          pl.BlockSpec((bn, d), lambda i: (i, 0)),
          pl.BlockSpec((bn, L), lambda i: (i, 0)),
      ],
      out_shape=[
          jax.ShapeDtypeStruct((npad, d), jnp.float32),
          jax.ShapeDtypeStruct((npad, L), jnp.float32),
      ],
  )(degp, x_pad)

  parts = _make_agg_kernel(npad, d, cpt)(xs, row4, col4)

  h_pad = pl.pallas_call(
      _head_kernel,
      grid=grid,
      in_specs=[
          pl.BlockSpec((bn, d), lambda i: (i, 0)),
          pl.BlockSpec((bn, d), lambda i: (i, 0)),
          pl.BlockSpec((bn, L), lambda i: (i, 0)),
          pl.BlockSpec((d, d), lambda i: (0, 0)),
          pl.BlockSpec((1, d), lambda i: (0, 0)),
      ],
      out_specs=pl.BlockSpec((bn, d), lambda i: (i, 0)),
      out_shape=jax.ShapeDtypeStruct((npad, d), jnp.float32),
  )(parts[:npad], parts[npad:], ds16, W, b.reshape(1, d))

  return h_pad[:n]
